# baseline (speedup 1.0000x reference)
import functools

import jax
import jax.numpy as jnp
from jax import lax
from jax.experimental import pallas as pl
from jax.experimental.pallas import tpu as pltpu
from jax.experimental.pallas import tpu_sc as plsc

_PLACEHOLDER_ID = 12345
_NW = 32
_L = 16


def kernel(tokenized_text, embedded_text, placeholder_emb):
    B, S, D = embedded_text.shape
    N = B * S
    per_w = N // _NW
    ngroups = per_w // _L
    tok = tokenized_text.reshape(N).astype(jnp.int32)
    mesh = plsc.VectorSubcoreMesh(core_axis_name="c", subcore_axis_name="s")

    @functools.partial(
        pl.kernel,
        out_type=(),
        mesh=mesh,
        compiler_params=pltpu.CompilerParams(needs_layout_passes=False),
        scratch_types=[
            pltpu.VMEM((per_w,), jnp.int32),
            pltpu.VMEM((D,), jnp.float32),
        ],
    )
    def run(tok_hbm, ph_hbm, buf, tok_v, ph_v):
        wid = lax.axis_index("s") * 2 + lax.axis_index("c")
        base = wid * per_w
        pltpu.sync_copy(tok_hbm.at[pl.ds(base, per_w)], tok_v)
        pltpu.sync_copy(ph_hbm, ph_v)
        lanes = lax.iota(jnp.int32, _L)

        def group(g, _):
            tokv = tok_v[pl.ds(g * _L, _L)]
            m0 = tokv == _PLACEHOLDER_ID
            cnt = plsc.all_reduce_population_count(m0)
            cnt_s = lax.reduce_max(cnt, (0,))

            @pl.when(cnt_s > 0)
            def _scatter():
                for l in range(_L):
                    t_l = lax.reduce_max(
                        jnp.where(lanes == l, tokv, jnp.int32(0)), (0,)
                    )

                    @pl.when(t_l == _PLACEHOLDER_ID)
                    def _one():
                        row = base + g * _L + l
                        b = row // S
                        s = row - b * S
                        pltpu.sync_copy(ph_v, buf.at[b, s])

            return _

        lax.fori_loop(0, ngroups, group, None)

    one = placeholder_emb[0] * jnp.float32(0.0) + jnp.float32(1.0)
    buf = jax.new_ref(embedded_text * one)
    run(tok, placeholder_emb, buf)
    return jax.freeze(buf)


# speedup vs baseline: 1.0595x; 1.0595x over previous
import functools

import jax
import jax.numpy as jnp
from jax import lax
from jax.experimental import pallas as pl
from jax.experimental.pallas import tpu as pltpu

_PLACEHOLDER_ID = 12345
_MAX_SPARSE = 128
_CB = 8


def _scatter_body(src_any, tok_any, ph_any, out_any, tok_v, ph_v, m_v, buf_v, sem, bsem):
    B, S = tok_any.shape
    D = ph_any.shape[0]

    pltpu.make_async_copy(tok_any, tok_v, sem).start()
    pltpu.make_async_copy(ph_any, ph_v, bsem).start()
    pltpu.make_async_copy(tok_any, tok_v, sem).wait()
    pltpu.make_async_copy(ph_any, ph_v, bsem).wait()

    tokm = tok_v[...]
    row_i = lax.broadcasted_iota(jnp.int32, (B, S), 0)
    col_i = lax.broadcasted_iota(jnp.int32, (B, S), 1)
    m = (tokm == _PLACEHOLDER_ID) & (col_i < S)
    total = jnp.sum(m.astype(jnp.int32))
    big = jnp.int32(2**30)
    m_v[...] = jnp.where(m, row_i * S + col_i, big)

    @pl.when((total > 0) & (total <= _MAX_SPARSE))
    def _sparse():
        def body(r0):
            b = r0 // S
            s = r0 - b * S
            pltpu.make_async_copy(ph_v, out_any.at[b, s], bsem).start()
            pltpu.make_async_copy(ph_v, out_any.at[b, s], bsem).wait()
            fl = m_v[...]
            fl = jnp.where(fl == r0, big, fl)
            m_v[...] = fl
            return jnp.min(fl)

        r_first = jnp.min(m_v[...])
        lax.while_loop(lambda r: r < big, body, r_first)

    @pl.when(total > _MAX_SPARSE)
    def _bulk():
        nchunk = B // _CB
        for c in range(nchunk):
            blk = out_any.at[pl.ds(c * _CB, _CB)]
            pltpu.make_async_copy(blk, buf_v, bsem).start()
            pltpu.make_async_copy(blk, buf_v, bsem).wait()
            tok3 = lax.broadcast_in_dim(
                tok_v[pl.ds(c * _CB, _CB), :], (_CB, S, D), (0, 1)
            )
            sel = tok3 == _PLACEHOLDER_ID
            buf_v[...] = jnp.where(sel, ph_v[...][None, None, :], buf_v[...])
            pltpu.make_async_copy(buf_v, blk, bsem).start()
            pltpu.make_async_copy(buf_v, blk, bsem).wait()


def kernel(tokenized_text, embedded_text, placeholder_emb):
    B, S, D = embedded_text.shape
    tok = tokenized_text.astype(jnp.int32)
    one = placeholder_emb[0] * jnp.float32(0.0) + jnp.float32(1.0)
    src = embedded_text * one
    out = pl.pallas_call(
        _scatter_body,
        in_specs=[
            pl.BlockSpec(memory_space=pl.ANY),
            pl.BlockSpec(memory_space=pl.ANY),
            pl.BlockSpec(memory_space=pl.ANY),
        ],
        out_specs=pl.BlockSpec(memory_space=pl.ANY),
        out_shape=jax.ShapeDtypeStruct((B, S, D), jnp.float32),
        input_output_aliases={0: 0},
        scratch_shapes=[
            pltpu.VMEM((B, S), jnp.int32),
            pltpu.VMEM((D,), jnp.float32),
            pltpu.VMEM((B, S), jnp.int32),
            pltpu.VMEM((_CB, S, D), jnp.float32),
            pltpu.SemaphoreType.DMA,
            pltpu.SemaphoreType.DMA,
        ],
    )(src, tok, placeholder_emb)
    return out


# speedup vs baseline: 1.0658x; 1.0059x over previous
import functools

import jax
import jax.numpy as jnp
from jax import lax
from jax.experimental import pallas as pl
from jax.experimental.pallas import tpu as pltpu

_PLACEHOLDER_ID = 12345
_MAX_SPARSE = 128
_CB = 8


def _scatter_body(src_any, tok_any, ph_any, out_any, tok_v, ph_v, m_v, buf_v, sem, bsem):
    B, S = tok_any.shape
    D = ph_any.shape[0]

    pltpu.make_async_copy(tok_any, tok_v, sem).start()
    pltpu.make_async_copy(ph_any, ph_v, bsem).start()
    pltpu.make_async_copy(tok_any, tok_v, sem).wait()
    pltpu.make_async_copy(ph_any, ph_v, bsem).wait()

    tokm = tok_v[...]
    row_i = lax.broadcasted_iota(jnp.int32, (B, S), 0)
    col_i = lax.broadcasted_iota(jnp.int32, (B, S), 1)
    m = (tokm == _PLACEHOLDER_ID) & (col_i < S)
    total = jnp.sum(m.astype(jnp.int32))
    big = jnp.int32(2**30)
    m_v[...] = jnp.where(m, row_i * S + col_i, big)

    @pl.when((total > 0) & (total <= _MAX_SPARSE))
    def _sparse():
        def body(r0):
            b = r0 // S
            s = r0 - b * S
            pltpu.make_async_copy(ph_v, out_any.at[b, s], bsem).start()
            pltpu.make_async_copy(ph_v, out_any.at[b, s], bsem).wait()
            fl = m_v[...]
            fl = jnp.where(fl == r0, big, fl)
            m_v[...] = fl
            return jnp.min(fl)

        r_first = jnp.min(m_v[...])
        lax.while_loop(lambda r: r < big, body, r_first)

    @pl.when(total > _MAX_SPARSE)
    def _bulk():
        nchunk = B // _CB
        for c in range(nchunk):
            blk = out_any.at[pl.ds(c * _CB, _CB)]
            pltpu.make_async_copy(blk, buf_v, bsem).start()
            pltpu.make_async_copy(blk, buf_v, bsem).wait()
            tok3 = lax.broadcast_in_dim(
                tok_v[pl.ds(c * _CB, _CB), :], (_CB, S, D), (0, 1)
            )
            sel = tok3 == _PLACEHOLDER_ID
            buf_v[...] = jnp.where(sel, ph_v[...][None, None, :], buf_v[...])
            pltpu.make_async_copy(buf_v, blk, bsem).start()
            pltpu.make_async_copy(buf_v, blk, bsem).wait()


def kernel(tokenized_text, embedded_text, placeholder_emb):
    B, S, D = embedded_text.shape
    tok = tokenized_text.astype(jnp.int32)
    src = embedded_text
    out = pl.pallas_call(
        _scatter_body,
        in_specs=[
            pl.BlockSpec(memory_space=pl.ANY),
            pl.BlockSpec(memory_space=pl.ANY),
            pl.BlockSpec(memory_space=pl.ANY),
        ],
        out_specs=pl.BlockSpec(memory_space=pl.ANY),
        out_shape=jax.ShapeDtypeStruct((B, S, D), jnp.float32),
        input_output_aliases={0: 0},
        scratch_shapes=[
            pltpu.VMEM((B, S), jnp.int32),
            pltpu.VMEM((D,), jnp.float32),
            pltpu.VMEM((B, S), jnp.int32),
            pltpu.VMEM((_CB, S, D), jnp.float32),
            pltpu.SemaphoreType.DMA,
            pltpu.SemaphoreType.DMA,
        ],
    )(src, tok, placeholder_emb)
    return out
